# baseline (speedup 1.0000x reference)
import functools

import jax
import jax.numpy as jnp
from jax import lax
from jax.experimental import pallas as pl
from jax.experimental.pallas import tpu as pltpu
from jax.experimental.pallas import tpu_sc as plsc

VOCAB_SIZE = 100000
EMB_DIM = 128

HEAD = 74400
BLK = 7440
NBLK = HEAD // BLK

TAIL = VOCAB_SIZE - HEAD
NW = 32
ROWS_PT = TAIL // NW
CHUNK = 160
NCHUNK = ROWS_PT // CHUNK
NBUF = 3
GROUPS = CHUNK // 16


def _permute(v, idx):
    return lax.gather(
        v,
        idx.reshape(16, 1),
        lax.GatherDimensionNumbers(
            offset_dims=(), collapsed_slice_dims=(0,), start_index_map=(0,)
        ),
        slice_sizes=(1,),
        mode=lax.GatherScatterMode.PROMISE_IN_BOUNDS,
    )

def _sc_tail_logits(idx, table, W):
    mesh = plsc.VectorSubcoreMesh(core_axis_name="c", subcore_axis_name="s")

    @functools.partial(
        pl.kernel,
        mesh=mesh,
        out_type=jax.ShapeDtypeStruct((NW, ROWS_PT), jnp.float32),
        scratch_types=[
            pltpu.VMEM((1,), jnp.int32),
            pltpu.VMEM((1, EMB_DIM), jnp.float32),
            pltpu.VMEM((NBUF, CHUNK, EMB_DIM), jnp.float32),
            pltpu.VMEM((ROWS_PT,), jnp.float32),
            pltpu.SemaphoreType.DMA,
            pltpu.SemaphoreType.DMA,
            pltpu.SemaphoreType.DMA,
            pltpu.SemaphoreType.DMA,
        ],
    )
    def k(idx_hbm, table_hbm, w_hbm, out_hbm, idx_v, e_v, wbuf_v, log_v,
          sem_e, sem0, sem1, sem2):
        c = lax.axis_index("c")
        s = lax.axis_index("s")
        wid = s * 2 + c
        base = HEAD + wid * ROWS_PT
        sems = [sem0, sem1, sem2]

        pltpu.sync_copy(idx_hbm, idx_v)
        pltpu.async_copy(table_hbm.at[idx_v], e_v, sem_e)

        def start(ch):
            pltpu.async_copy(
                w_hbm.at[pl.ds(base + ch * CHUNK, CHUNK)],
                wbuf_v.at[ch % NBUF],
                sems[ch % NBUF],
            )

        for ch in range(min(NBUF, NCHUNK)):
            start(ch)

        pltpu.make_async_copy(table_hbm.at[idx_v], e_v, sem_e).wait()
        e_regs = [e_v[0, pl.ds(16 * j, 16)] for j in range(8)]
        lanes = lax.iota(jnp.int32, 16)
        perms = [jnp.bitwise_xor(lanes, k) for k in (1, 2, 4, 8)]

        for ch in range(NCHUNK):
            buf = ch % NBUF
            pltpu.make_async_copy(
                w_hbm.at[pl.ds(base + ch * CHUNK, CHUNK)],
                wbuf_v.at[buf],
                sems[buf],
            ).wait()

            def group_body(g, _, buf=buf, ch=ch):
                staged = jnp.zeros((16,), jnp.float32)
                for rl in range(16):
                    row = g * 16 + rl
                    acc = wbuf_v[buf, row, pl.ds(0, 16)] * e_regs[0]
                    for j in range(1, 8):
                        acc = acc + wbuf_v[buf, row, pl.ds(16 * j, 16)] * e_regs[j]
                    for p in perms:
                        acc = acc + _permute(acc, p)
                    staged = jnp.where(lanes == rl, acc, staged)
                log_v[pl.ds(ch * CHUNK + g * 16, 16)] = staged
                return 0

            lax.fori_loop(0, GROUPS, group_body, 0)

            if ch + NBUF < NCHUNK:
                start(ch + NBUF)

        pltpu.sync_copy(log_v, out_hbm.at[wid])

    return k(idx, table, W)


def _tc1_body(idx_ref, e_ref, w_ref, out_ref):
    e = e_ref[0]
    out_ref[0] = lax.dot_general(
        e, w_ref[...], (((1,), (1,)), ((), ())), preferred_element_type=jnp.float32
    )


def _tc1_head_logits(idx, emb_table, W):
    grid_spec = pltpu.PrefetchScalarGridSpec(
        num_scalar_prefetch=1,
        grid=(NBLK,),
        in_specs=[
            pl.BlockSpec((1, 1, EMB_DIM), lambda i, idx_ref: (idx_ref[0], 0, 0)),
            pl.BlockSpec((BLK, EMB_DIM), lambda i, idx_ref: (i, 0)),
        ],
        out_specs=pl.BlockSpec((1, 1, BLK), lambda i, idx_ref: (i, 0, 0)),
    )
    return pl.pallas_call(
        _tc1_body,
        grid_spec=grid_spec,
        out_shape=jax.ShapeDtypeStruct((NBLK, 1, BLK), jnp.float32),
    )(idx, emb_table.reshape(VOCAB_SIZE, 1, EMB_DIM), W)


def _tc2_body(h_ref, t_ref, bh_ref, bt_ref, oh_ref, ot_ref):
    xh = h_ref[...] + bh_ref[...]
    xt = t_ref[...] + bt_ref[...]
    m = jnp.maximum(jnp.max(xh), jnp.max(xt))
    s = jnp.sum(jnp.exp(xh - m)) + jnp.sum(jnp.exp(xt - m))
    lse = m + jnp.log(s)
    oh_ref[...] = xh - lse
    ot_ref[...] = xt - lse


def _tc2_logsoftmax(h, t, bh, bt):
    return pl.pallas_call(
        _tc2_body,
        out_shape=(
            jax.ShapeDtypeStruct((93, 800), jnp.float32),
            jax.ShapeDtypeStruct((NW, ROWS_PT), jnp.float32),
        ),
    )(h, t, bh, bt)


def kernel(inputs, emb_table, W, b):
    idx = inputs.astype(jnp.int32)
    tail = _sc_tail_logits(idx, emb_table, W)
    head = _tc1_head_logits(idx, emb_table, W)
    bh = lax.slice(b, (0,), (HEAD,)).reshape(93, 800)
    bt = lax.slice(b, (HEAD,), (VOCAB_SIZE,)).reshape(NW, ROWS_PT)
    oh, ot = _tc2_logsoftmax(head.reshape(93, 800), tail, bh, bt)
    return jnp.concatenate(
        [oh.reshape(1, HEAD), ot.reshape(1, TAIL)], axis=1
    )


# speedup vs baseline: 1.0464x; 1.0464x over previous
import functools

import jax
import jax.numpy as jnp
from jax import lax
from jax.experimental import pallas as pl
from jax.experimental.pallas import tpu as pltpu
from jax.experimental.pallas import tpu_sc as plsc

VOCAB_SIZE = 100000
EMB_DIM = 128

HEAD = 74400
BLK = 7440
NBLK = HEAD // BLK

TAIL = VOCAB_SIZE - HEAD
NW = 32
ROWS_PT = TAIL // NW
CHUNK = 160
NCHUNK = ROWS_PT // CHUNK
NBUF = 3
GROUPS = CHUNK // 16


def _permute(v, idx):
    return lax.gather(
        v,
        idx.reshape(16, 1),
        lax.GatherDimensionNumbers(
            offset_dims=(), collapsed_slice_dims=(0,), start_index_map=(0,)
        ),
        slice_sizes=(1,),
        mode=lax.GatherScatterMode.PROMISE_IN_BOUNDS,
    )

def _sc_tail_logits(idx, table, W):
    mesh = plsc.VectorSubcoreMesh(core_axis_name="c", subcore_axis_name="s")

    @functools.partial(
        pl.kernel,
        mesh=mesh,
        out_type=jax.ShapeDtypeStruct((NW, ROWS_PT), jnp.float32),
        scratch_types=[
            pltpu.VMEM((1,), jnp.int32),
            pltpu.VMEM((1, EMB_DIM), jnp.float32),
            pltpu.VMEM((NBUF, CHUNK, EMB_DIM), jnp.float32),
            pltpu.VMEM((ROWS_PT,), jnp.float32),
            pltpu.SemaphoreType.DMA,
            pltpu.SemaphoreType.DMA,
            pltpu.SemaphoreType.DMA,
            pltpu.SemaphoreType.DMA,
        ],
    )
    def k(idx_hbm, table_hbm, w_hbm, out_hbm, idx_v, e_v, wbuf_v, log_v,
          sem_e, sem0, sem1, sem2):
        c = lax.axis_index("c")
        s = lax.axis_index("s")
        wid = s * 2 + c
        base = HEAD + wid * ROWS_PT
        sems = [sem0, sem1, sem2]

        pltpu.sync_copy(idx_hbm, idx_v)
        pltpu.async_copy(table_hbm.at[idx_v], e_v, sem_e)

        def start(ch):
            pltpu.async_copy(
                w_hbm.at[pl.ds(base + ch * CHUNK, CHUNK)],
                wbuf_v.at[ch % NBUF],
                sems[ch % NBUF],
            )

        for ch in range(min(NBUF, NCHUNK)):
            start(ch)

        pltpu.make_async_copy(table_hbm.at[idx_v], e_v, sem_e).wait()
        e_regs = [e_v[0, pl.ds(16 * j, 16)] for j in range(8)]
        lanes = lax.iota(jnp.int32, 16)
        perms = [jnp.bitwise_xor(lanes, k) for k in (1, 2, 4, 8)]

        for ch in range(NCHUNK):
            buf = ch % NBUF
            pltpu.make_async_copy(
                w_hbm.at[pl.ds(base + ch * CHUNK, CHUNK)],
                wbuf_v.at[buf],
                sems[buf],
            ).wait()

            def group_body(g, _, buf=buf, ch=ch):
                staged = jnp.zeros((16,), jnp.float32)
                for rl in range(16):
                    row = g * 16 + rl
                    acc = wbuf_v[buf, row, pl.ds(0, 16)] * e_regs[0]
                    for j in range(1, 8):
                        acc = acc + wbuf_v[buf, row, pl.ds(16 * j, 16)] * e_regs[j]
                    for p in perms:
                        acc = acc + _permute(acc, p)
                    staged = jnp.where(lanes == rl, acc, staged)
                log_v[pl.ds(ch * CHUNK + g * 16, 16)] = staged
                return 0

            lax.fori_loop(0, GROUPS, group_body, 0)

            if ch + NBUF < NCHUNK:
                start(ch + NBUF)

        pltpu.sync_copy(log_v, out_hbm.at[wid])

    return k(idx, table, W)


def _tc1_body(idx_ref, e_ref, w_ref, out_ref):
    e = e_ref[0]
    out_ref[0] = lax.dot_general(
        e, w_ref[...], (((1,), (1,)), ((), ())), preferred_element_type=jnp.float32
    )


def _tc1_head_logits(idx, emb_table, W):
    grid_spec = pltpu.PrefetchScalarGridSpec(
        num_scalar_prefetch=1,
        grid=(NBLK,),
        in_specs=[
            pl.BlockSpec((1, 1, EMB_DIM), lambda i, idx_ref: (idx_ref[0], 0, 0)),
            pl.BlockSpec((BLK, EMB_DIM), lambda i, idx_ref: (i, 0)),
        ],
        out_specs=pl.BlockSpec((1, 1, BLK), lambda i, idx_ref: (i, 0, 0)),
    )
    return pl.pallas_call(
        _tc1_body,
        grid_spec=grid_spec,
        out_shape=jax.ShapeDtypeStruct((NBLK, 1, BLK), jnp.float32),
    )(idx, emb_table.reshape(VOCAB_SIZE, 1, EMB_DIM), W)


HROWS = HEAD // 800


def _tc2_body(h_ref, t_ref, b_ref, o_ref):
    xh = h_ref[...] + b_ref[0:HROWS]
    xt = t_ref[...] + b_ref[HROWS:125].reshape(NW, ROWS_PT)
    m = jnp.maximum(jnp.max(xh), jnp.max(xt))
    s = jnp.sum(jnp.exp(xh - m)) + jnp.sum(jnp.exp(xt - m))
    lse = m + jnp.log(s)
    o_ref[0:HROWS] = xh - lse
    o_ref[HROWS:125] = (xt - lse).reshape(32, 800)


def _tc2_logsoftmax(h, t, b2):
    return pl.pallas_call(
        _tc2_body,
        out_shape=jax.ShapeDtypeStruct((125, 800), jnp.float32),
    )(h, t, b2)


def kernel(inputs, emb_table, W, b):
    idx = inputs.astype(jnp.int32)
    head = _tc1_head_logits(idx, emb_table, W)
    tail = _sc_tail_logits(idx, emb_table, W)
    out = _tc2_logsoftmax(head.reshape(HROWS, 800), tail, b.reshape(125, 800))
    return out.reshape(1, VOCAB_SIZE)


# speedup vs baseline: 1.1641x; 1.1125x over previous
import functools

import jax
import jax.numpy as jnp
from jax import lax
from jax.experimental import pallas as pl
from jax.experimental.pallas import tpu as pltpu
from jax.experimental.pallas import tpu_sc as plsc

VOCAB_SIZE = 100000
EMB_DIM = 128

HEAD = 74400
BLK = 7440
NBLK = HEAD // BLK

TAIL = VOCAB_SIZE - HEAD
NW = 32
ROWS_PT = TAIL // NW
CHUNK = 160
NCHUNK = ROWS_PT // CHUNK
GROUPS = CHUNK // 16


def _permute(v, idx):
    return lax.gather(
        v,
        idx.reshape(16, 1),
        lax.GatherDimensionNumbers(
            offset_dims=(), collapsed_slice_dims=(0,), start_index_map=(0,)
        ),
        slice_sizes=(1,),
        mode=lax.GatherScatterMode.PROMISE_IN_BOUNDS,
    )


def _sc_tail_logits(idx, table, W, b):
    mesh = plsc.VectorSubcoreMesh(core_axis_name="c", subcore_axis_name="s")

    @functools.partial(
        pl.kernel,
        mesh=mesh,
        out_type=jax.ShapeDtypeStruct((NW, ROWS_PT), jnp.float32),
        scratch_types=[
            pltpu.VMEM((1,), jnp.int32),
            pltpu.VMEM((1, EMB_DIM), jnp.float32),
            pltpu.VMEM((ROWS_PT, EMB_DIM), jnp.float32),
            pltpu.VMEM((ROWS_PT,), jnp.float32),
            pltpu.VMEM((ROWS_PT,), jnp.float32),
            pltpu.SemaphoreType.DMA,
            pltpu.SemaphoreType.DMA,
        ],
    )
    def k(idx_hbm, table_hbm, w_hbm, b_hbm, out_hbm,
          idx_v, e_v, wbuf_v, b_v, log_v, sem_e, sem_w):
        c = lax.axis_index("c")
        s = lax.axis_index("s")
        wid = s * 2 + c
        base = HEAD + wid * ROWS_PT

        pltpu.sync_copy(idx_hbm, idx_v)
        pltpu.async_copy(table_hbm.at[idx_v], e_v, sem_e)

        for ch in range(NCHUNK):
            pltpu.async_copy(
                w_hbm.at[pl.ds(base + ch * CHUNK, CHUNK)],
                wbuf_v.at[pl.ds(ch * CHUNK, CHUNK)],
                sem_w,
            )
        pltpu.sync_copy(b_hbm.at[pl.ds(base, ROWS_PT)], b_v)

        pltpu.make_async_copy(table_hbm.at[idx_v], e_v, sem_e).wait()
        e_regs = [e_v[0, pl.ds(16 * j, 16)] for j in range(8)]
        lanes = lax.iota(jnp.int32, 16)
        perms = [jnp.bitwise_xor(lanes, k) for k in (1, 2, 4, 8)]

        def chunk_body(ch, _):
            pltpu.make_async_copy(
                w_hbm.at[pl.ds(base, CHUNK)],
                wbuf_v.at[pl.ds(0, CHUNK)],
                sem_w,
            ).wait()

            def group_body(g, _):
                staged = jnp.zeros((16,), jnp.float32)
                for rl in range(16):
                    row = g * 16 + rl
                    acc = wbuf_v[row, pl.ds(0, 16)] * e_regs[0]
                    for j in range(1, 8):
                        acc = acc + wbuf_v[row, pl.ds(16 * j, 16)] * e_regs[j]
                    for p in perms:
                        acc = acc + _permute(acc, p)
                    staged = jnp.where(lanes == rl, acc, staged)
                log_v[pl.ds(g * 16, 16)] = staged + b_v[pl.ds(g * 16, 16)]
                return 0

            lax.fori_loop(ch * GROUPS, (ch + 1) * GROUPS, group_body, 0,
                          unroll=False)
            return 0

        lax.fori_loop(0, NCHUNK, chunk_body, 0, unroll=False)
        pltpu.sync_copy(log_v, out_hbm.at[wid])

    return k(idx, table, W, b)


def _tc1_body(idx_ref, e_ref, w_ref, out_ref):
    e = e_ref[0]
    out_ref[0] = lax.dot_general(
        e, w_ref[...], (((1,), (1,)), ((), ())), preferred_element_type=jnp.float32
    )


def _tc1_head_logits(idx, emb_table, W):
    grid_spec = pltpu.PrefetchScalarGridSpec(
        num_scalar_prefetch=1,
        grid=(NBLK,),
        in_specs=[
            pl.BlockSpec((1, 1, EMB_DIM), lambda i, idx_ref: (idx_ref[0], 0, 0)),
            pl.BlockSpec((BLK, EMB_DIM), lambda i, idx_ref: (i, 0)),
        ],
        out_specs=pl.BlockSpec((1, 1, BLK), lambda i, idx_ref: (i, 0, 0)),
    )
    return pl.pallas_call(
        _tc1_body,
        grid_spec=grid_spec,
        out_shape=jax.ShapeDtypeStruct((NBLK, 1, BLK), jnp.float32),
    )(idx, emb_table.reshape(VOCAB_SIZE, 1, EMB_DIM), W)


def _tc2_body(h_ref, t_ref, b_ref, o_ref):
    xt = t_ref[...]
    m = jnp.max(xt)
    for i in range(NBLK):
        xi = h_ref[i] + b_ref[0:1, pl.ds(i * BLK, BLK)]
        m = jnp.maximum(m, jnp.max(xi))
    s = jnp.sum(jnp.exp(xt - m))
    for i in range(NBLK):
        xi = h_ref[i] + b_ref[0:1, pl.ds(i * BLK, BLK)]
        s = s + jnp.sum(jnp.exp(xi - m))
    lse = m + jnp.log(s)
    for i in range(NBLK):
        xi = h_ref[i] + b_ref[0:1, pl.ds(i * BLK, BLK)]
        o_ref[0:1, pl.ds(i * BLK, BLK)] = xi - lse
    for r in range(NW):
        o_ref[0:1, pl.ds(HEAD + r * ROWS_PT, ROWS_PT)] = (
            t_ref[r] - lse
        ).reshape(1, ROWS_PT)


def _tc2_logsoftmax(h, t, b2):
    return pl.pallas_call(
        _tc2_body,
        out_shape=jax.ShapeDtypeStruct((1, VOCAB_SIZE), jnp.float32),
    )(h, t, b2)


def kernel(inputs, emb_table, W, b):
    idx = inputs.astype(jnp.int32)
    head = _tc1_head_logits(idx, emb_table, W)
    tail = _sc_tail_logits(idx, emb_table, W, b)
    return _tc2_logsoftmax(head, tail, b.reshape(1, VOCAB_SIZE))


# speedup vs baseline: 1.8790x; 1.6141x over previous
import functools

import jax
import jax.numpy as jnp
from jax import lax
from jax.experimental import pallas as pl
from jax.experimental.pallas import tpu as pltpu
from jax.experimental.pallas import tpu_sc as plsc

VOCAB_SIZE = 100000
EMB_DIM = 128
BLK = 10000
NBLK = VOCAB_SIZE // BLK


def _sc_gather(idx, table):
    mesh = plsc.VectorSubcoreMesh(core_axis_name="c", subcore_axis_name="s")

    @functools.partial(
        pl.kernel,
        mesh=mesh,
        out_type=jax.ShapeDtypeStruct((1, EMB_DIM), jnp.float32),
        scratch_types=[
            pltpu.VMEM((1,), jnp.int32),
            pltpu.VMEM((1, EMB_DIM), jnp.float32),
            pltpu.SemaphoreType.DMA,
        ],
    )
    def k(idx_hbm, table_hbm, out_hbm, idx_v, row_v, sem):
        c = lax.axis_index("c")
        s = lax.axis_index("s")

        @pl.when((c == 0) & (s == 0))
        def _():
            pltpu.sync_copy(idx_hbm, idx_v)
            pltpu.async_copy(table_hbm.at[idx_v], row_v, sem).wait()
            pltpu.sync_copy(row_v, out_hbm)

    return k(idx, table)


def _tc_body(idx_ref, e_ref, w_ref, b_ref, out_ref):
    i = pl.program_id(0)

    e = e_ref[0]
    logits = lax.dot_general(
        e, w_ref[...], (((1,), (1,)), ((), ())), preferred_element_type=jnp.float32
    )
    out_ref[pl.ds(i, 1), :] = logits

    @pl.when(i == NBLK - 1)
    def _():
        x = out_ref[...] + b_ref[...]
        m = jnp.max(x)
        lse = m + jnp.log(jnp.sum(jnp.exp(x - m)))
        out_ref[...] = x - lse


def _tc_linear_logsoftmax(idx, emb_table, W, b):
    grid_spec = pltpu.PrefetchScalarGridSpec(
        num_scalar_prefetch=1,
        grid=(NBLK,),
        in_specs=[
            pl.BlockSpec((1, 1, EMB_DIM), lambda i, idx_ref: (idx_ref[0], 0, 0)),
            pl.BlockSpec((BLK, EMB_DIM), lambda i, idx_ref: (i, 0)),
            pl.BlockSpec((NBLK, BLK), lambda i, idx_ref: (0, 0)),
        ],
        out_specs=pl.BlockSpec((NBLK, BLK), lambda i, idx_ref: (0, 0)),
    )
    return pl.pallas_call(
        _tc_body,
        grid_spec=grid_spec,
        out_shape=jax.ShapeDtypeStruct((NBLK, BLK), jnp.float32),
    )(idx, emb_table.reshape(VOCAB_SIZE, 1, EMB_DIM), W, b.reshape(NBLK, BLK))


def kernel(inputs, emb_table, W, b):
    idx = inputs.astype(jnp.int32)
    out = _tc_linear_logsoftmax(idx, emb_table, W, b)
    return out.reshape(1, VOCAB_SIZE)


# speedup vs baseline: 1.9485x; 1.0370x over previous
import jax
import jax.numpy as jnp
from jax import lax
from jax.experimental import pallas as pl
from jax.experimental.pallas import tpu as pltpu

VOCAB_SIZE = 100000
EMB_DIM = 128
BLK = 10000
NBLK = VOCAB_SIZE // BLK


def _tc_body(idx_ref, e_ref, w_ref, b_ref, out_ref):
    i = pl.program_id(0)

    e = e_ref[0]
    logits = lax.dot_general(
        e, w_ref[...], (((1,), (1,)), ((), ())), preferred_element_type=jnp.float32
    )
    out_ref[pl.ds(i, 1), :] = logits

    @pl.when(i == NBLK - 1)
    def _():
        x = out_ref[...] + b_ref[...]
        m = jnp.max(x)
        lse = m + jnp.log(jnp.sum(jnp.exp(x - m)))
        out_ref[...] = x - lse


def _tc_linear_logsoftmax(idx, emb_table, W, b):
    grid_spec = pltpu.PrefetchScalarGridSpec(
        num_scalar_prefetch=1,
        grid=(NBLK,),
        in_specs=[
            pl.BlockSpec((1, 1, EMB_DIM), lambda i, idx_ref: (idx_ref[0], 0, 0)),
            pl.BlockSpec((BLK, EMB_DIM), lambda i, idx_ref: (i, 0)),
            pl.BlockSpec((NBLK, BLK), lambda i, idx_ref: (0, 0)),
        ],
        out_specs=pl.BlockSpec((NBLK, BLK), lambda i, idx_ref: (0, 0)),
    )
    return pl.pallas_call(
        _tc_body,
        grid_spec=grid_spec,
        out_shape=jax.ShapeDtypeStruct((NBLK, BLK), jnp.float32),
    )(idx, emb_table.reshape(VOCAB_SIZE, 1, EMB_DIM), W, b.reshape(NBLK, BLK))


def kernel(inputs, emb_table, W, b):
    idx = inputs.astype(jnp.int32)
    out = _tc_linear_logsoftmax(idx, emb_table, W, b)
    return out.reshape(1, VOCAB_SIZE)
